# baseline (speedup 1.0000x reference)
import functools

import jax
import jax.numpy as jnp
from jax import lax
from jax.experimental import pallas as pl
from jax.experimental.pallas import tpu as pltpu

_G = 256
_BN = 2000


def _body(batch_ref, x_ref, W1_ref, b1_ref, W2_ref, b2_ref, Wt_ref, bt_ref,
          out_ref, c_ref, z_ref, p_ref):
    i = pl.program_id(0)
    nb = pl.num_programs(0)

    @pl.when(i == 0)
    def _init():
        c_ref[...] = jnp.zeros_like(c_ref)
        z_ref[...] = jnp.zeros_like(z_ref)
        p_ref[...] = jnp.zeros_like(p_ref)

    xb_bf = x_ref[...].astype(jnp.bfloat16)
    h = jnp.tanh(
        jnp.dot(xb_bf, W1_ref[...].astype(jnp.bfloat16),
                preferred_element_type=jnp.float32)
        + b1_ref[...])
    sT = lax.dot_general(W2_ref[...], h, (((0,), (1,)), ((), ())),
                         preferred_element_type=jnp.float32)
    sT = sT + b2_ref[0, 0]

    c_old = c_ref[...]
    c_new = jnp.maximum(c_old, jnp.max(sT, axis=1, keepdims=True))
    alpha = jnp.exp(c_old - c_new)
    eT = jnp.exp(sT - c_new)

    ids = batch_ref[0]
    gi = lax.broadcasted_iota(jnp.int32, (_G, sT.shape[1]), 0)
    E = jnp.where(gi == ids, jnp.broadcast_to(eT, gi.shape),
                  0.0).astype(jnp.bfloat16)

    ones = jnp.ones((sT.shape[1], 1), jnp.bfloat16)
    z_ref[...] = z_ref[...] * alpha + jnp.dot(
        E, ones, preferred_element_type=jnp.float32)
    p_ref[...] = p_ref[...] * alpha + jnp.dot(
        E, xb_bf, preferred_element_type=jnp.float32)
    c_ref[...] = c_new

    @pl.when(i == nb - 1)
    def _fin():
        z = z_ref[...]
        pooled = jnp.where(z > 0.0, p_ref[...] / z, 0.0)
        out_ref[...] = jnp.dot(
            pooled, Wt_ref[...], preferred_element_type=jnp.float32
        ) + bt_ref[...]


@jax.jit
def kernel(x, batch, W1, b1, W2, b2, Wt, bt):
    N, D = x.shape
    DH = W1.shape[1]
    DO = Wt.shape[1]

    nb = -(-N // _BN)
    Np = nb * _BN
    if Np != N:
        x = jnp.pad(x, ((0, Np - N), (0, 0)))
        batch = jnp.pad(batch.astype(jnp.int32), (0, Np - N),
                        constant_values=_G)
    batch3 = batch.astype(jnp.int32).reshape(nb, 1, _BN)
    b1r = b1.reshape(1, DH).astype(jnp.float32)
    b2r = b2.reshape(1, 1).astype(jnp.float32)
    btr = bt.reshape(1, DO).astype(jnp.float32)

    out = pl.pallas_call(
        _body,
        grid=(nb,),
        in_specs=[
            pl.BlockSpec((1, 1, _BN), lambda i: (i, 0, 0)),
            pl.BlockSpec((_BN, D), lambda i: (i, 0)),
            pl.BlockSpec((D, DH), lambda i: (0, 0)),
            pl.BlockSpec((1, DH), lambda i: (0, 0)),
            pl.BlockSpec((DH, 1), lambda i: (0, 0)),
            pl.BlockSpec((1, 1), lambda i: (0, 0)),
            pl.BlockSpec((D, DO), lambda i: (0, 0)),
            pl.BlockSpec((1, DO), lambda i: (0, 0)),
        ],
        out_specs=pl.BlockSpec((_G, DO), lambda i: (0, 0)),
        out_shape=jax.ShapeDtypeStruct((_G, DO), jnp.float32),
        scratch_shapes=[
            pltpu.VMEM((1, 1), jnp.float32),
            pltpu.VMEM((_G, 1), jnp.float32),
            pltpu.VMEM((_G, D), jnp.float32),
        ],
        compiler_params=pltpu.CompilerParams(
            dimension_semantics=("arbitrary",)),
    )(batch3, x, W1, b1r, W2, b2r, Wt, btr)
    return out


# speedup vs baseline: 1.2077x; 1.2077x over previous
import functools

import jax
import jax.numpy as jnp
from jax import lax
from jax.experimental import pallas as pl
from jax.experimental.pallas import tpu as pltpu

_G = 256
_BN = 2000


def _body(batch_ref, x_ref, W1_ref, b1_ref, W2_ref, b2_ref, Wt_ref, bt_ref,
          out_ref, c_ref, z_ref, p_ref):
    i = pl.program_id(0)
    nb = pl.num_programs(0)

    @pl.when(i == 0)
    def _init():
        c_ref[0] = 0.0
        z_ref[...] = jnp.zeros_like(z_ref)
        p_ref[...] = jnp.zeros_like(p_ref)

    xb_bf = x_ref[...].astype(jnp.bfloat16)
    h = jnp.tanh(
        jnp.dot(xb_bf, W1_ref[...].astype(jnp.bfloat16),
                preferred_element_type=jnp.float32)
        + b1_ref[...])
    sT = lax.dot_general(W2_ref[...], h, (((0,), (1,)), ((), ())),
                         preferred_element_type=jnp.float32)
    sT = sT + b2_ref[0, 0]

    c_old = c_ref[0]
    c_new = jnp.maximum(c_old, jnp.max(sT))
    eT = jnp.exp(sT - c_new)

    ids = batch_ref[0]
    gi = lax.broadcasted_iota(jnp.int32, (_G, sT.shape[1]), 0)
    Ef = jnp.where(gi == ids, jnp.broadcast_to(eT, gi.shape), 0.0)

    @pl.when(c_new > c_old)
    def _rescale():
        alpha = jnp.exp(jnp.full((1, 1), c_old - c_new, jnp.float32))
        z_ref[...] = z_ref[...] * alpha
        p_ref[...] = p_ref[...] * alpha

    z_ref[...] = z_ref[...] + jnp.sum(Ef, axis=1, keepdims=True)
    p_ref[...] = p_ref[...] + jnp.dot(
        Ef.astype(jnp.bfloat16), xb_bf,
        preferred_element_type=jnp.float32)
    c_ref[0] = c_new

    @pl.when(i == nb - 1)
    def _fin():
        z = z_ref[...]
        pooled = jnp.where(z > 0.0, p_ref[...] / z, 0.0)
        out_ref[...] = jnp.dot(
            pooled, Wt_ref[...], preferred_element_type=jnp.float32
        ) + bt_ref[...]


@jax.jit
def kernel(x, batch, W1, b1, W2, b2, Wt, bt):
    N, D = x.shape
    DH = W1.shape[1]
    DO = Wt.shape[1]

    nb = -(-N // _BN)
    Np = nb * _BN
    if Np != N:
        x = jnp.pad(x, ((0, Np - N), (0, 0)))
        batch = jnp.pad(batch.astype(jnp.int32), (0, Np - N),
                        constant_values=_G)
    batch3 = batch.astype(jnp.int32).reshape(nb, 1, _BN)
    b1r = b1.reshape(1, DH).astype(jnp.float32)
    b2r = b2.reshape(1, 1).astype(jnp.float32)
    btr = bt.reshape(1, DO).astype(jnp.float32)

    out = pl.pallas_call(
        _body,
        grid=(nb,),
        in_specs=[
            pl.BlockSpec((1, 1, _BN), lambda i: (i, 0, 0)),
            pl.BlockSpec((_BN, D), lambda i: (i, 0)),
            pl.BlockSpec((D, DH), lambda i: (0, 0)),
            pl.BlockSpec((1, DH), lambda i: (0, 0)),
            pl.BlockSpec((DH, 1), lambda i: (0, 0)),
            pl.BlockSpec((1, 1), lambda i: (0, 0)),
            pl.BlockSpec((D, DO), lambda i: (0, 0)),
            pl.BlockSpec((1, DO), lambda i: (0, 0)),
        ],
        out_specs=pl.BlockSpec((_G, DO), lambda i: (0, 0)),
        out_shape=jax.ShapeDtypeStruct((_G, DO), jnp.float32),
        scratch_shapes=[
            pltpu.SMEM((1,), jnp.float32),
            pltpu.VMEM((_G, 1), jnp.float32),
            pltpu.VMEM((_G, D), jnp.float32),
        ],
        compiler_params=pltpu.CompilerParams(
            dimension_semantics=("arbitrary",)),
    )(batch3, x, W1, b1r, W2, b2r, Wt, btr)
    return out
